# baseline (speedup 1.0000x reference)
import functools

import jax
import jax.numpy as jnp
from jax import lax
from jax.experimental import pallas as pl
from jax.experimental.pallas import tpu as pltpu
from jax.experimental.pallas import tpu_sc as plsc

F = 64
CTX = 24
IN_DIM = 3 * F + CTX


@functools.lru_cache(maxsize=None)
def _build_sc_forward(B, n_info, n_shift):
    NC, NS = 2, 16
    NW = NC * NS
    CHUNK = B // NW
    NG = CHUNK // 16
    NIDX = 128
    KC = CHUNK // NIDX

    mesh = plsc.VectorSubcoreMesh(core_axis_name="c", subcore_axis_name="s")

    @functools.partial(
        pl.kernel,
        mesh=mesh,
        out_type=jax.ShapeDtypeStruct((B,), jnp.float32),
        scratch_types=[
            pltpu.VMEM((KC, NIDX), jnp.int32),
            pltpu.VMEM((KC, NIDX), jnp.int32),
            pltpu.VMEM((CHUNK,), jnp.int32),
            pltpu.VMEM((CHUNK, F), jnp.float32),
            pltpu.VMEM((CHUNK, F), jnp.float32),
            pltpu.VMEM((n_shift, F), jnp.float32),
            pltpu.VMEM((CHUNK, n_info), jnp.float32),
            pltpu.VMEM((CHUNK,), jnp.float32),
            pltpu.VMEM((CHUNK,), jnp.float32),
            pltpu.VMEM((IN_DIM + 8,), jnp.float32),
            pltpu.VMEM((CHUNK,), jnp.float32),
            pltpu.SemaphoreType.DMA,
        ],
        compiler_params=pltpu.CompilerParams(
            needs_layout_passes=False, use_tc_tiling_on_sc=False),
    )
    def sc_forward(uid2, sid, iid2, info, visits, buys, utab, itab, stab,
                   wb, out, idx_u, idx_i, sid_v, rows_u, rows_i, stab_v,
                   info_v, vis_v, buy_v, w_v, out_v, sem):
        wid = lax.axis_index("s") * NC + lax.axis_index("c")
        base = wid * CHUNK

        pltpu.sync_copy(uid2.at[pl.ds(wid * KC, KC)], idx_u)
        pltpu.sync_copy(iid2.at[pl.ds(wid * KC, KC)], idx_i)

        copies = []
        for k in range(KC):
            copies.append(pltpu.async_copy(
                utab.at[idx_u.at[k]], rows_u.at[pl.ds(k * NIDX, NIDX)], sem))
        for k in range(KC):
            copies.append(pltpu.async_copy(
                itab.at[idx_i.at[k]], rows_i.at[pl.ds(k * NIDX, NIDX)], sem))

        pltpu.sync_copy(sid.at[pl.ds(base, CHUNK)], sid_v)
        pltpu.sync_copy(stab, stab_v)
        pltpu.sync_copy(wb, w_v)
        pltpu.sync_copy(info.at[pl.ds(base, CHUNK)], info_v)
        pltpu.sync_copy(visits.at[pl.ds(base, CHUNK)], vis_v)
        pltpu.sync_copy(buys.at[pl.ds(base, CHUNK)], buy_v)

        for c in copies:
            c.wait()

        lane = lax.iota(jnp.int32, 16)
        wvecs = [w_v[pl.ds(c * 16, 16)] for c in range((IN_DIM + 8) // 16)]

        def wscal(j):
            return wvecs[j // 16][j % 16]

        bias = wscal(IN_DIM)

        @pl.loop(0, NG)
        def _group(g):
            off = pl.multiple_of(g * 16, 16)
            row_idx = off + lane
            acc = bias + jnp.zeros((16,), jnp.float32)
            for j in range(n_info):
                col = plsc.load_gather(
                    info_v, [row_idx, jnp.full((16,), j, jnp.int32)])
                acc = acc + col * wscal(j)
            acc = acc + vis_v[pl.ds(off, 16)] * wscal(n_info)
            acc = acc + buy_v[pl.ds(off, 16)] * wscal(n_info + 1)
            sid_vec = sid_v[pl.ds(off, 16)]
            for j in range(F):
                col = plsc.load_gather(
                    stab_v, [sid_vec, jnp.full((16,), j, jnp.int32)])
                acc = acc + col * wscal(CTX + j)
            for j in range(F):
                col = plsc.load_gather(
                    rows_u, [row_idx, jnp.full((16,), j, jnp.int32)])
                acc = acc + col * wscal(CTX + F + j)
            for j in range(F):
                col = plsc.load_gather(
                    rows_i, [row_idx, jnp.full((16,), j, jnp.int32)])
                acc = acc + col * wscal(CTX + 2 * F + j)
            out_v[pl.ds(off, 16)] = 1.0 / (1.0 + jnp.exp(-acc))

        pltpu.sync_copy(out_v, out.at[pl.ds(base, CHUNK)])

    return sc_forward


def kernel(user_ids, shift_ids, item_ids, category, info, visits, buys,
           user_table, item_table, shift_table, W, b):
    del category
    B = user_ids.shape[0]
    uid2 = user_ids.astype(jnp.int32).reshape(B // 128, 128)
    iid2 = item_ids.astype(jnp.int32).reshape(B // 128, 128)
    sid = shift_ids.astype(jnp.int32)
    wb = jnp.concatenate([
        W.reshape(-1).astype(jnp.float32),
        b.reshape(-1).astype(jnp.float32),
        jnp.zeros((7,), jnp.float32),
    ])
    fwd = _build_sc_forward(B, info.shape[1], shift_table.shape[0])
    out = fwd(uid2, sid, iid2, info, visits, buys,
              user_table, item_table, shift_table, wb)
    return out.reshape(B, 1)


# speedup vs baseline: 1.5176x; 1.5176x over previous
import functools

import jax
import jax.numpy as jnp
from jax import lax
from jax.experimental import pallas as pl
from jax.experimental.pallas import tpu as pltpu
from jax.experimental.pallas import tpu_sc as plsc

F = 64
CTX = 24
IN_DIM = 3 * F + CTX


@functools.lru_cache(maxsize=None)
def _build_sc_forward(B, n_info, n_shift):
    NC, NS = 2, 16
    NW = NC * NS
    CHUNK = B // NW
    NG = CHUNK // 16

    mesh = plsc.VectorSubcoreMesh(core_axis_name="c", subcore_axis_name="s")

    @functools.partial(
        pl.kernel,
        mesh=mesh,
        out_type=jax.ShapeDtypeStruct((B,), jnp.float32),
        scratch_types=[
            pltpu.VMEM((CHUNK,), jnp.int32),
            pltpu.VMEM((CHUNK,), jnp.int32),
            pltpu.VMEM((CHUNK,), jnp.int32),
            pltpu.VMEM((CHUNK, F), jnp.float32),
            pltpu.VMEM((n_shift * F,), jnp.float32),
            pltpu.VMEM((CHUNK * n_info,), jnp.float32),
            pltpu.VMEM((CHUNK,), jnp.float32),
            pltpu.VMEM((CHUNK,), jnp.float32),
            pltpu.VMEM((IN_DIM + 8,), jnp.float32),
            pltpu.VMEM((CHUNK,), jnp.float32),
            pltpu.SemaphoreType.DMA,
        ],
        compiler_params=pltpu.CompilerParams(needs_layout_passes=False),
    )
    def sc_forward(uid, sid, iid, info1, visits, buys, utab, itab, stab1,
                   wb, out, uid_v, iid_v, sid_v, rows, stab_v,
                   info_v, vis_v, buy_v, w_v, out_v, sem):
        wid = lax.axis_index("s") * NC + lax.axis_index("c")
        base = wid * CHUNK

        pltpu.sync_copy(uid.at[pl.ds(base, CHUNK)], uid_v)
        pltpu.sync_copy(iid.at[pl.ds(base, CHUNK)], iid_v)

        def fire_rows(idx_ref, tab):
            @pl.loop(0, NG)
            def _fire(g):
                goff = pl.multiple_of(g * 16, 16)
                vec = idx_ref[pl.ds(goff, 16)]
                for l in range(16):
                    pltpu.async_copy(
                        tab.at[pl.ds(vec[l], 1)],
                        rows.at[pl.ds(goff + l, 1)], sem)

        def drain_rows(tab):
            @pl.loop(0, NG)
            def _drain(g):
                goff = pl.multiple_of(g * 16, 16)
                for l in range(16):
                    pltpu.make_async_copy(
                        tab.at[pl.ds(0, 1)],
                        rows.at[pl.ds(goff + l, 1)], sem).wait()

        fire_rows(uid_v, utab)

        pltpu.sync_copy(sid.at[pl.ds(base, CHUNK)], sid_v)
        pltpu.sync_copy(stab1, stab_v)
        pltpu.sync_copy(wb, w_v)
        pltpu.sync_copy(info1.at[pl.ds(base * n_info, CHUNK * n_info)], info_v)
        pltpu.sync_copy(visits.at[pl.ds(base, CHUNK)], vis_v)
        pltpu.sync_copy(buys.at[pl.ds(base, CHUNK)], buy_v)

        drain_rows(utab)

        lane = lax.iota(jnp.int32, 16)
        wvecs = [w_v[pl.ds(c * 16, 16)] for c in range((IN_DIM + 8) // 16)]

        def wscal(j):
            return wvecs[j // 16][j % 16]

        bias = wscal(IN_DIM)

        @pl.loop(0, NG)
        def _user(g):
            off = pl.multiple_of(g * 16, 16)
            row_idx = off + lane
            acc = jnp.zeros((16,), jnp.float32)
            for j in range(F):
                col = plsc.load_gather(
                    rows, [row_idx, jnp.full((16,), j, jnp.int32)])
                acc = acc + col * wscal(CTX + F + j)
            out_v[pl.ds(off, 16)] = acc

        fire_rows(iid_v, itab)
        drain_rows(itab)

        @pl.loop(0, NG)
        def _rest(g):
            off = pl.multiple_of(g * 16, 16)
            row_idx = off + lane
            ibase = row_idx * n_info
            sbase = sid_v[pl.ds(off, 16)] * F
            acc = out_v[pl.ds(off, 16)] + bias
            for j in range(n_info):
                acc = acc + plsc.load_gather(info_v, [ibase + j]) * wscal(j)
            acc = acc + vis_v[pl.ds(off, 16)] * wscal(n_info)
            acc = acc + buy_v[pl.ds(off, 16)] * wscal(n_info + 1)
            for j in range(F):
                acc = acc + plsc.load_gather(stab_v, [sbase + j]) * wscal(CTX + j)
            for j in range(F):
                col = plsc.load_gather(
                    rows, [row_idx, jnp.full((16,), j, jnp.int32)])
                acc = acc + col * wscal(CTX + 2 * F + j)
            out_v[pl.ds(off, 16)] = 1.0 / (1.0 + jnp.exp(-acc))

        pltpu.sync_copy(out_v, out.at[pl.ds(base, CHUNK)])

    return sc_forward


def kernel(user_ids, shift_ids, item_ids, category, info, visits, buys,
           user_table, item_table, shift_table, W, b):
    del category
    B = user_ids.shape[0]
    uid = user_ids.astype(jnp.int32)
    iid = item_ids.astype(jnp.int32)
    sid = shift_ids.astype(jnp.int32)
    info1 = info.reshape(-1)
    stab1 = shift_table.reshape(-1)
    wb = jnp.concatenate([
        W.reshape(-1).astype(jnp.float32),
        b.reshape(-1).astype(jnp.float32),
        jnp.zeros((7,), jnp.float32),
    ])
    fwd = _build_sc_forward(B, info.shape[1], shift_table.shape[0])
    out = fwd(uid, sid, iid, info1, visits, buys,
              user_table, item_table, stab1, wb)
    return out.reshape(B, 1)


# speedup vs baseline: 3.5225x; 2.3211x over previous
import functools

import jax
import jax.numpy as jnp
from jax import lax
from jax.experimental import pallas as pl
from jax.experimental.pallas import tpu as pltpu
from jax.experimental.pallas import tpu_sc as plsc

F = 64
CTX = 24
IN_DIM = 3 * F + CTX
BLK = 8192


def _rowdot_body(w_ref, m_ref, o_ref):
    prod = jax.lax.dot_general(
        w_ref[...], m_ref[...], (((1,), (0,)), ((), ())),
        preferred_element_type=jnp.float32,
        precision=jax.lax.Precision.HIGHEST)
    o_ref[...] = prod.reshape(o_ref.shape)


@functools.lru_cache(maxsize=None)
def _build_rowdot(K, N):
    nblk = -(-N // BLK)

    return pl.pallas_call(
        _rowdot_body,
        grid=(nblk,),
        in_specs=[
            pl.BlockSpec((1, K), lambda j: (0, 0)),
            pl.BlockSpec((K, BLK), lambda j: (0, j)),
        ],
        out_specs=pl.BlockSpec((BLK,), lambda j: (j,)),
        out_shape=jax.ShapeDtypeStruct((nblk * BLK,), jnp.float32),
    )


@functools.lru_cache(maxsize=None)
def _build_sc_combine(B, n_shift):
    NC, NS = 2, 16
    NW = NC * NS
    CHUNK = B // NW
    NG = CHUNK // 16

    mesh = plsc.VectorSubcoreMesh(core_axis_name="c", subcore_axis_name="s")

    @functools.partial(
        pl.kernel,
        mesh=mesh,
        out_type=jax.ShapeDtypeStruct((B,), jnp.float32),
        scratch_types=[
            pltpu.VMEM((CHUNK,), jnp.int32),
            pltpu.VMEM((CHUNK,), jnp.int32),
            pltpu.VMEM((CHUNK,), jnp.int32),
            pltpu.VMEM((CHUNK,), jnp.float32),
            pltpu.VMEM((CHUNK,), jnp.float32),
            pltpu.VMEM((n_shift * F,), jnp.float32),
            pltpu.VMEM((16,), jnp.float32),
            pltpu.VMEM((CHUNK,), jnp.float32),
            pltpu.VMEM((CHUNK,), jnp.float32),
            pltpu.VMEM((CHUNK,), jnp.float32),
            pltpu.VMEM((IN_DIM + 8,), jnp.float32),
            pltpu.VMEM((CHUNK,), jnp.float32),
            pltpu.SemaphoreType.DMA,
        ],
        compiler_params=pltpu.CompilerParams(needs_layout_passes=False),
    )
    def sc_combine(uid, iid, sid, du, di, ctxd, visits, buys, stab1, wb,
                   out, uid_v, iid_v, sid_v, dug_v, dig_v, stab_v, sd_v,
                   ctx_v, vis_v, buy_v, w_v, out_v, sem):
        wid = lax.axis_index("s") * NC + lax.axis_index("c")
        base = wid * CHUNK

        pltpu.sync_copy(uid.at[pl.ds(base, CHUNK)], uid_v)
        pltpu.sync_copy(iid.at[pl.ds(base, CHUNK)], iid_v)

        copies = []
        for g in range(NG):
            off = g * 16
            copies.append(pltpu.async_copy(
                du.at[uid_v[pl.ds(off, 16)]], dug_v.at[pl.ds(off, 16)], sem))
            copies.append(pltpu.async_copy(
                di.at[iid_v[pl.ds(off, 16)]], dig_v.at[pl.ds(off, 16)], sem))

        pltpu.sync_copy(sid.at[pl.ds(base, CHUNK)], sid_v)
        pltpu.sync_copy(stab1, stab_v)
        pltpu.sync_copy(wb, w_v)
        pltpu.sync_copy(ctxd.at[pl.ds(base, CHUNK)], ctx_v)
        pltpu.sync_copy(visits.at[pl.ds(base, CHUNK)], vis_v)
        pltpu.sync_copy(buys.at[pl.ds(base, CHUNK)], buy_v)

        lane = lax.iota(jnp.int32, 16)
        wvecs = [w_v[pl.ds(c * 16, 16)] for c in range((IN_DIM + 8) // 16)]

        def wscal(j):
            return wvecs[j // 16][j % 16]

        bias = wscal(IN_DIM)

        sdots = jnp.zeros((16,), jnp.float32)
        for s in range(n_shift):
            t = jnp.zeros((16,), jnp.float32)
            for k in range(F // 16):
                row = stab_v[pl.ds(s * F + k * 16, 16)]
                wsv = w_v[pl.ds(CTX + k * 16, 16)]
                t = t + row * wsv
            tot = lax.reduce_sum_p.bind(t, axes=(0,))
            sdots = jnp.where(lane == s, tot, sdots)
        sd_v[...] = sdots

        for c in copies:
            c.wait()

        @pl.loop(0, NG)
        def _group(g):
            off = pl.multiple_of(g * 16, 16)
            sval = plsc.load_gather(sd_v, [sid_v[pl.ds(off, 16)]])
            acc = (bias + dug_v[pl.ds(off, 16)] + dig_v[pl.ds(off, 16)]
                   + sval + ctx_v[pl.ds(off, 16)]
                   + vis_v[pl.ds(off, 16)] * wscal(22)
                   + buy_v[pl.ds(off, 16)] * wscal(23))
            out_v[pl.ds(off, 16)] = 1.0 / (1.0 + jnp.exp(-acc))

        pltpu.sync_copy(out_v, out.at[pl.ds(base, CHUNK)])

    return sc_combine


def kernel(user_ids, shift_ids, item_ids, category, info, visits, buys,
           user_table, item_table, shift_table, W, b):
    del category
    B = user_ids.shape[0]
    uid = user_ids.astype(jnp.int32)
    iid = item_ids.astype(jnp.int32)
    sid = shift_ids.astype(jnp.int32)
    n_info = info.shape[1]
    wc = W[:, :n_info]
    wu = W[:, CTX + F:CTX + 2 * F]
    wi = W[:, CTX + 2 * F:CTX + 3 * F]
    stab1 = shift_table.reshape(-1)
    wb = jnp.concatenate([
        W.reshape(-1).astype(jnp.float32),
        b.reshape(-1).astype(jnp.float32),
        jnp.zeros((7,), jnp.float32),
    ])
    du = _build_rowdot(F, user_table.shape[0])(wu, user_table.T)
    di = _build_rowdot(F, item_table.shape[0])(wi, item_table.T)
    ctxd = _build_rowdot(n_info, B)(wc, info.T)
    fwd = _build_sc_combine(B, shift_table.shape[0])
    out = fwd(uid, iid, sid, du, di, ctxd, visits, buys, stab1, wb)
    return out.reshape(B, 1)


# speedup vs baseline: 4.6022x; 1.3065x over previous
import functools

import jax
import jax.numpy as jnp
from jax import lax
from jax.experimental import pallas as pl
from jax.experimental.pallas import tpu as pltpu
from jax.experimental.pallas import tpu_sc as plsc

F = 64
CTX = 24
IN_DIM = 3 * F + CTX


def _rowdot_body(w_ref, m_ref, o_ref):
    prod = jax.lax.dot_general(
        w_ref[...], m_ref[...], (((1,), (0,)), ((), ())),
        preferred_element_type=jnp.float32,
        precision=jax.lax.Precision.HIGHEST)
    o_ref[...] = prod.reshape(o_ref.shape)


@functools.lru_cache(maxsize=None)
def _build_rowdot(K, N, blk):
    nblk = -(-N // blk)

    return pl.pallas_call(
        _rowdot_body,
        grid=(nblk,),
        in_specs=[
            pl.BlockSpec((1, K), lambda j: (0, 0)),
            pl.BlockSpec((K, blk), lambda j: (0, j)),
        ],
        out_specs=pl.BlockSpec((blk,), lambda j: (j,)),
        out_shape=jax.ShapeDtypeStruct((nblk * blk,), jnp.float32),
    )


@functools.lru_cache(maxsize=None)
def _build_sc_combine(B, n_shift):
    NC, NS = 2, 16
    NW = NC * NS
    CHUNK = B // NW
    NG = CHUNK // 16

    mesh = plsc.VectorSubcoreMesh(core_axis_name="c", subcore_axis_name="s")

    @functools.partial(
        pl.kernel,
        mesh=mesh,
        out_type=jax.ShapeDtypeStruct((B,), jnp.float32),
        scratch_types=[
            pltpu.VMEM((CHUNK,), jnp.int32),
            pltpu.VMEM((CHUNK,), jnp.int32),
            pltpu.VMEM((CHUNK,), jnp.int32),
            pltpu.VMEM((CHUNK,), jnp.float32),
            pltpu.VMEM((CHUNK,), jnp.float32),
            pltpu.VMEM((n_shift * F,), jnp.float32),
            pltpu.VMEM((16,), jnp.float32),
            pltpu.VMEM((CHUNK,), jnp.float32),
            pltpu.VMEM((CHUNK,), jnp.float32),
            pltpu.VMEM((CHUNK,), jnp.float32),
            pltpu.VMEM((IN_DIM + 8,), jnp.float32),
            pltpu.VMEM((CHUNK,), jnp.float32),
            pltpu.SemaphoreType.DMA,
        ],
        compiler_params=pltpu.CompilerParams(needs_layout_passes=False),
    )
    def sc_combine(uid, iid, sid, du, di, ctxd, visits, buys, stab1, wb,
                   out, uid_v, iid_v, sid_v, dug_v, dig_v, stab_v, sd_v,
                   ctx_v, vis_v, buy_v, w_v, out_v, sem):
        wid = lax.axis_index("s") * NC + lax.axis_index("c")
        base = wid * CHUNK

        pltpu.sync_copy(uid.at[pl.ds(base, CHUNK)], uid_v)
        pltpu.sync_copy(iid.at[pl.ds(base, CHUNK)], iid_v)

        copies = []
        for g in range(NG):
            off = g * 16
            copies.append(pltpu.async_copy(
                du.at[uid_v[pl.ds(off, 16)]], dug_v.at[pl.ds(off, 16)], sem))
            copies.append(pltpu.async_copy(
                di.at[iid_v[pl.ds(off, 16)]], dig_v.at[pl.ds(off, 16)], sem))

        pltpu.sync_copy(sid.at[pl.ds(base, CHUNK)], sid_v)
        pltpu.sync_copy(stab1, stab_v)
        pltpu.sync_copy(wb, w_v)
        pltpu.sync_copy(ctxd.at[pl.ds(base, CHUNK)], ctx_v)
        pltpu.sync_copy(visits.at[pl.ds(base, CHUNK)], vis_v)
        pltpu.sync_copy(buys.at[pl.ds(base, CHUNK)], buy_v)

        lane = lax.iota(jnp.int32, 16)
        wvecs = [w_v[pl.ds(c * 16, 16)] for c in range((IN_DIM + 8) // 16)]

        def wscal(j):
            return wvecs[j // 16][j % 16]

        bias = wscal(IN_DIM)

        sdots = jnp.zeros((16,), jnp.float32)
        for s in range(n_shift):
            t = jnp.zeros((16,), jnp.float32)
            for k in range(F // 16):
                row = stab_v[pl.ds(s * F + k * 16, 16)]
                wsv = w_v[pl.ds(CTX + k * 16, 16)]
                t = t + row * wsv
            tot = lax.reduce_sum_p.bind(t, axes=(0,))
            sdots = jnp.where(lane == s, tot, sdots)
        sd_v[...] = sdots

        for c in copies:
            c.wait()

        @pl.loop(0, NG)
        def _group(g):
            off = pl.multiple_of(g * 16, 16)
            sval = plsc.load_gather(sd_v, [sid_v[pl.ds(off, 16)]])
            acc = (bias + dug_v[pl.ds(off, 16)] + dig_v[pl.ds(off, 16)]
                   + sval + ctx_v[pl.ds(off, 16)]
                   + vis_v[pl.ds(off, 16)] * wscal(22)
                   + buy_v[pl.ds(off, 16)] * wscal(23))
            out_v[pl.ds(off, 16)] = 1.0 / (1.0 + jnp.exp(-acc))

        pltpu.sync_copy(out_v, out.at[pl.ds(base, CHUNK)])

    return sc_combine


def kernel(user_ids, shift_ids, item_ids, category, info, visits, buys,
           user_table, item_table, shift_table, W, b):
    del category
    B = user_ids.shape[0]
    uid = user_ids.astype(jnp.int32)
    iid = item_ids.astype(jnp.int32)
    sid = shift_ids.astype(jnp.int32)
    n_info = info.shape[1]
    wc = W[:, :n_info]
    wu = W[:, CTX + F:CTX + 2 * F]
    wi = W[:, CTX + 2 * F:CTX + 3 * F]
    stab1 = shift_table.reshape(-1)
    wb = jnp.concatenate([
        W.reshape(-1).astype(jnp.float32),
        b.reshape(-1).astype(jnp.float32),
        jnp.zeros((7,), jnp.float32),
    ])
    du = _build_rowdot(F, user_table.shape[0], 32768)(wu, user_table.T)
    di = _build_rowdot(F, item_table.shape[0], 32768)(wi, item_table.T)
    ctxd = _build_rowdot(n_info, B, B)(wc, info.T)
    fwd = _build_sc_combine(B, shift_table.shape[0])
    out = fwd(uid, iid, sid, du, di, ctxd, visits, buys, stab1, wb)
    return out.reshape(B, 1)


# speedup vs baseline: 6.2126x; 1.3499x over previous
import functools

import jax
import jax.numpy as jnp
from jax import lax
from jax.experimental import pallas as pl
from jax.experimental.pallas import tpu as pltpu
from jax.experimental.pallas import tpu_sc as plsc

F = 64
CTX = 24
IN_DIM = 3 * F + CTX


def _rowdot_body(w_ref, m_ref, o_ref):
    prod = jax.lax.dot_general(
        w_ref[...], m_ref[...], (((1,), (0,)), ((), ())),
        preferred_element_type=jnp.float32,
        precision=jax.lax.Precision.DEFAULT)
    o_ref[...] = prod.reshape(o_ref.shape)


@functools.lru_cache(maxsize=None)
def _build_rowdot(K, N, blk):
    nblk = -(-N // blk)

    return pl.pallas_call(
        _rowdot_body,
        grid=(nblk,),
        in_specs=[
            pl.BlockSpec((1, K), lambda j: (0, 0)),
            pl.BlockSpec((K, blk), lambda j: (0, j)),
        ],
        out_specs=pl.BlockSpec((blk,), lambda j: (j,)),
        out_shape=jax.ShapeDtypeStruct((nblk * blk,), jnp.float32),
    )


@functools.lru_cache(maxsize=None)
def _build_sc_combine(B, n_shift):
    NC, NS = 2, 16
    NW = NC * NS
    CHUNK = B // NW
    NG = CHUNK // 16

    mesh = plsc.VectorSubcoreMesh(core_axis_name="c", subcore_axis_name="s")

    @functools.partial(
        pl.kernel,
        mesh=mesh,
        out_type=jax.ShapeDtypeStruct((B,), jnp.float32),
        scratch_types=[
            pltpu.VMEM((CHUNK,), jnp.int32),
            pltpu.VMEM((CHUNK,), jnp.int32),
            pltpu.VMEM((CHUNK,), jnp.int32),
            pltpu.VMEM((CHUNK,), jnp.float32),
            pltpu.VMEM((CHUNK,), jnp.float32),
            pltpu.VMEM((n_shift * F,), jnp.float32),
            pltpu.VMEM((16,), jnp.float32),
            pltpu.VMEM((CHUNK,), jnp.float32),
            pltpu.VMEM((CHUNK,), jnp.float32),
            pltpu.VMEM((CHUNK,), jnp.float32),
            pltpu.VMEM((IN_DIM + 8,), jnp.float32),
            pltpu.VMEM((CHUNK,), jnp.float32),
            pltpu.SemaphoreType.DMA,
        ],
        compiler_params=pltpu.CompilerParams(needs_layout_passes=False),
    )
    def sc_combine(uid, iid, sid, du, di, ctxd, visits, buys, stab1, wb,
                   out, uid_v, iid_v, sid_v, dug_v, dig_v, stab_v, sd_v,
                   ctx_v, vis_v, buy_v, w_v, out_v, sem):
        wid = lax.axis_index("s") * NC + lax.axis_index("c")
        base = wid * CHUNK

        pltpu.sync_copy(uid.at[pl.ds(base, CHUNK)], uid_v)
        pltpu.sync_copy(iid.at[pl.ds(base, CHUNK)], iid_v)

        copies = []
        for g in range(NG):
            off = g * 16
            copies.append(pltpu.async_copy(
                du.at[uid_v[pl.ds(off, 16)]], dug_v.at[pl.ds(off, 16)], sem))
            copies.append(pltpu.async_copy(
                di.at[iid_v[pl.ds(off, 16)]], dig_v.at[pl.ds(off, 16)], sem))

        pltpu.sync_copy(sid.at[pl.ds(base, CHUNK)], sid_v)
        pltpu.sync_copy(stab1, stab_v)
        pltpu.sync_copy(wb, w_v)
        pltpu.sync_copy(ctxd.at[pl.ds(base, CHUNK)], ctx_v)
        pltpu.sync_copy(visits.at[pl.ds(base, CHUNK)], vis_v)
        pltpu.sync_copy(buys.at[pl.ds(base, CHUNK)], buy_v)

        lane = lax.iota(jnp.int32, 16)
        wvecs = [w_v[pl.ds(c * 16, 16)] for c in range((IN_DIM + 8) // 16)]

        def wscal(j):
            return wvecs[j // 16][j % 16]

        bias = wscal(IN_DIM)

        sdots = jnp.zeros((16,), jnp.float32)
        for s in range(n_shift):
            t = jnp.zeros((16,), jnp.float32)
            for k in range(F // 16):
                row = stab_v[pl.ds(s * F + k * 16, 16)]
                wsv = w_v[pl.ds(CTX + k * 16, 16)]
                t = t + row * wsv
            tot = lax.reduce_sum_p.bind(t, axes=(0,))
            sdots = jnp.where(lane == s, tot, sdots)
        sd_v[...] = sdots

        for c in copies:
            c.wait()

        @pl.loop(0, NG)
        def _group(g):
            off = pl.multiple_of(g * 16, 16)
            sval = plsc.load_gather(sd_v, [sid_v[pl.ds(off, 16)]])
            acc = (bias + dug_v[pl.ds(off, 16)] + dig_v[pl.ds(off, 16)]
                   + sval + ctx_v[pl.ds(off, 16)]
                   + vis_v[pl.ds(off, 16)] * wscal(22)
                   + buy_v[pl.ds(off, 16)] * wscal(23))
            out_v[pl.ds(off, 16)] = 1.0 / (1.0 + jnp.exp(-acc))

        pltpu.sync_copy(out_v, out.at[pl.ds(base, CHUNK)])

    return sc_combine


def kernel(user_ids, shift_ids, item_ids, category, info, visits, buys,
           user_table, item_table, shift_table, W, b):
    del category
    B = user_ids.shape[0]
    uid = user_ids.astype(jnp.int32)
    iid = item_ids.astype(jnp.int32)
    sid = shift_ids.astype(jnp.int32)
    n_info = info.shape[1]
    wc = W[:, :n_info]
    wu = W[:, CTX + F:CTX + 2 * F]
    wi = W[:, CTX + 2 * F:CTX + 3 * F]
    stab1 = shift_table.reshape(-1)
    wb = jnp.concatenate([
        W.reshape(-1).astype(jnp.float32),
        b.reshape(-1).astype(jnp.float32),
        jnp.zeros((7,), jnp.float32),
    ])
    du = _build_rowdot(F, user_table.shape[0], 32768)(wu, user_table.T)
    di = _build_rowdot(F, item_table.shape[0], 32768)(wi, item_table.T)
    ctxd = _build_rowdot(n_info, B, B)(wc, info.T)
    fwd = _build_sc_combine(B, shift_table.shape[0])
    out = fwd(uid, iid, sid, du, di, ctxd, visits, buys, stab1, wb)
    return out.reshape(B, 1)
